# baseline (speedup 1.0000x reference)
import jax
import jax.numpy as jnp
from jax import lax
from jax.experimental import pallas as pl
from jax.experimental.pallas import tpu as pltpu
from jax.experimental.pallas import tpu_sc as plsc

_B = 16384
_V = 100000
_D = 32
_NCAT = 8
_NCONT = 4
_EMB_D = _NCAT * _D

_NC = 2
_NS = 16
_NW = _NC * _NS
_BPW = _B // _NW
_RC = 256
_NCHUNK = _BPW // _RC


def _body(*refs):
    cats = refs[0:_NCAT]
    tabs = refs[_NCAT:2 * _NCAT]
    out = refs[2 * _NCAT]
    idx_v, rows_v, sem = refs[2 * _NCAT + 1:]

    wid = lax.axis_index("s") * _NC + lax.axis_index("c")
    base0 = wid * _BPW

    for ch in range(_NCHUNK):
        base = base0 + ch * _RC
        for t in range(_NCAT):
            pltpu.sync_copy(cats[t].at[pl.ds(base, _RC)], idx_v.at[t])
        gathers = [
            pltpu.async_copy(tabs[t].at[idx_v.at[t]], rows_v.at[t], sem)
            for t in range(_NCAT)
        ]
        for t in range(_NCAT):
            gathers[t].wait()
            pltpu.sync_copy(
                rows_v.at[t],
                out.at[pl.ds(base, _RC), pl.ds(t * _D, _D)])


_sc_call = pl.kernel(
    _body,
    mesh=plsc.VectorSubcoreMesh(core_axis_name="c", subcore_axis_name="s"),
    out_type=jax.ShapeDtypeStruct((_B, _EMB_D), jnp.float32),
    scratch_types=[
        pltpu.VMEM((_NCAT, _RC), jnp.int32),
        pltpu.VMEM((_NCAT, _RC, _D), jnp.float32),
        pltpu.SemaphoreType.DMA,
    ],
    compiler_params=pltpu.CompilerParams(use_tc_tiling_on_sc=False),
)


def kernel(cat_0, cat_1, cat_2, cat_3, cat_4, cat_5, cat_6, cat_7,
           table_0, table_1, table_2, table_3, table_4, table_5, table_6,
           table_7, cont_0, cont_1, cont_2, cont_3):
    cats = [c.reshape(_B).astype(jnp.int32)
            for c in (cat_0, cat_1, cat_2, cat_3, cat_4, cat_5, cat_6, cat_7)]
    tabs = (table_0, table_1, table_2, table_3, table_4, table_5, table_6,
            table_7)
    emb = _sc_call(*cats, *tabs)
    cont = jnp.stack(
        [c.astype(jnp.float32) for c in (cont_0, cont_1, cont_2, cont_3)],
        axis=-1)
    return jnp.concatenate([cont, emb], axis=-1)
